# baseline (speedup 1.0000x reference)
import functools

import jax
import jax.numpy as jnp
from jax import lax
from jax.experimental import pallas as pl
from jax.experimental.pallas import tpu as pltpu
from jax.experimental.pallas import tpu_sc as plsc

VOCAB = 100000
EMB = 32
HIDDEN = 512
CTX = 20
BATCH = 1024

BN = 1024


def _sc_gather(table, idx_flat, n_rows):
    info = plsc.get_sparse_core_info()
    nw = info.num_cores * info.num_subcores
    b_per_w = n_rows // nw
    mesh = plsc.VectorSubcoreMesh(core_axis_name="c", subcore_axis_name="s")

    @functools.partial(
        pl.kernel,
        mesh=mesh,
        compiler_params=pltpu.CompilerParams(use_tc_tiling_on_sc=False),
        out_type=jax.ShapeDtypeStruct((n_rows, EMB), jnp.float32),
        scratch_types=[
            pltpu.VMEM((b_per_w,), jnp.int32),
            pltpu.VMEM((b_per_w, EMB), jnp.float32),
            pltpu.SemaphoreType.DMA,
        ],
    )
    def gather_k(idx_hbm, table_hbm, out_hbm, idx_v, rows_v, sem):
        wid = lax.axis_index("s") * info.num_cores + lax.axis_index("c")
        base = wid * b_per_w
        pltpu.sync_copy(idx_hbm.at[pl.ds(base, b_per_w)], idx_v)
        pltpu.async_copy(table_hbm.at[idx_v], rows_v, sem).wait()
        pltpu.sync_copy(rows_v, out_hbm.at[pl.ds(base, b_per_w)])

    return gather_k(idx_flat, table)


def _mlp1_body(flat_ref, w1_ref, b1_ref, h_ref):
    a = flat_ref[...].astype(jnp.bfloat16)
    w = w1_ref[...].astype(jnp.bfloat16)
    h = jnp.dot(a, w, preferred_element_type=jnp.float32)
    h_ref[...] = jnp.maximum(h + b1_ref[...], 0.0).astype(jnp.bfloat16)


def _mlp2_body(h_ref, w2_ref, b2_ref, out_ref):
    w = w2_ref[...].astype(jnp.bfloat16)
    acc = jnp.dot(h_ref[...], w, preferred_element_type=jnp.float32)
    out_ref[...] = acc + b2_ref[...]


def kernel(x, emb_table, W1, b1, W2, b2):
    idx_flat = x.reshape(-1).astype(jnp.int32)
    flat = _sc_gather(emb_table, idx_flat, BATCH * CTX)
    flat = flat.reshape(BATCH, CTX * EMB)

    h = pl.pallas_call(
        _mlp1_body,
        out_shape=jax.ShapeDtypeStruct((BATCH, HIDDEN), jnp.bfloat16),
    )(flat, W1, b1.reshape(1, HIDDEN))

    nblocks = pl.cdiv(VOCAB, BN)
    logits = pl.pallas_call(
        _mlp2_body,
        grid=(nblocks,),
        in_specs=[
            pl.BlockSpec((BATCH, HIDDEN), lambda j: (0, 0)),
            pl.BlockSpec((HIDDEN, BN), lambda j: (0, j)),
            pl.BlockSpec((1, BN), lambda j: (0, j)),
        ],
        out_specs=pl.BlockSpec((BATCH, BN), lambda j: (0, j)),
        out_shape=jax.ShapeDtypeStruct((BATCH, VOCAB), jnp.float32),
        compiler_params=pltpu.CompilerParams(
            dimension_semantics=("parallel",),
        ),
    )(h, W2, b2.reshape(1, VOCAB))

    return logits


# speedup vs baseline: 1.0341x; 1.0341x over previous
import functools

import jax
import jax.numpy as jnp
from jax import lax
from jax.experimental import pallas as pl
from jax.experimental.pallas import tpu as pltpu
from jax.experimental.pallas import tpu_sc as plsc

VOCAB = 100000
EMB = 32
HIDDEN = 512
CTX = 20
BATCH = 1024

BN = 4096


def _sc_gather(table, idx_flat, n_rows):
    info = plsc.get_sparse_core_info()
    nw = info.num_cores * info.num_subcores
    b_per_w = n_rows // nw
    mesh = plsc.VectorSubcoreMesh(core_axis_name="c", subcore_axis_name="s")

    @functools.partial(
        pl.kernel,
        mesh=mesh,
        compiler_params=pltpu.CompilerParams(use_tc_tiling_on_sc=False),
        out_type=jax.ShapeDtypeStruct((n_rows, EMB), jnp.float32),
        scratch_types=[
            pltpu.VMEM((b_per_w,), jnp.int32),
            pltpu.VMEM((b_per_w, EMB), jnp.float32),
            pltpu.SemaphoreType.DMA,
        ],
    )
    def gather_k(idx_hbm, table_hbm, out_hbm, idx_v, rows_v, sem):
        wid = lax.axis_index("s") * info.num_cores + lax.axis_index("c")
        base = wid * b_per_w
        pltpu.sync_copy(idx_hbm.at[pl.ds(base, b_per_w)], idx_v)
        pltpu.async_copy(table_hbm.at[idx_v], rows_v, sem).wait()
        pltpu.sync_copy(rows_v, out_hbm.at[pl.ds(base, b_per_w)])

    return gather_k(idx_flat, table)


def _mlp1_body(flat_ref, w1_ref, b1_ref, h_ref):
    a = flat_ref[...].astype(jnp.bfloat16)
    w = w1_ref[...].astype(jnp.bfloat16)
    h = jnp.dot(a, w, preferred_element_type=jnp.float32)
    h_ref[...] = jnp.maximum(h + b1_ref[...], 0.0).astype(jnp.bfloat16)


def _mlp2_body(h_ref, w2_ref, b2_ref, out_ref):
    w = w2_ref[...].astype(jnp.bfloat16)
    acc = jnp.dot(h_ref[...], w, preferred_element_type=jnp.float32)
    out_ref[...] = acc + b2_ref[...]


def kernel(x, emb_table, W1, b1, W2, b2):
    idx_flat = x.reshape(-1).astype(jnp.int32)
    flat = _sc_gather(emb_table, idx_flat, BATCH * CTX)
    flat = flat.reshape(BATCH, CTX * EMB)

    h = pl.pallas_call(
        _mlp1_body,
        out_shape=jax.ShapeDtypeStruct((BATCH, HIDDEN), jnp.bfloat16),
    )(flat, W1, b1.reshape(1, HIDDEN))

    nblocks = pl.cdiv(VOCAB, BN)
    logits = pl.pallas_call(
        _mlp2_body,
        grid=(nblocks,),
        in_specs=[
            pl.BlockSpec((BATCH, HIDDEN), lambda j: (0, 0)),
            pl.BlockSpec((HIDDEN, BN), lambda j: (0, j)),
            pl.BlockSpec((1, BN), lambda j: (0, j)),
        ],
        out_specs=pl.BlockSpec((BATCH, BN), lambda j: (0, j)),
        out_shape=jax.ShapeDtypeStruct((BATCH, VOCAB), jnp.float32),
        compiler_params=pltpu.CompilerParams(
            dimension_semantics=("parallel",),
        ),
    )(h, W2, b2.reshape(1, VOCAB))

    return logits


# speedup vs baseline: 1.0370x; 1.0028x over previous
import functools

import jax
import jax.numpy as jnp
from jax import lax
from jax.experimental import pallas as pl
from jax.experimental.pallas import tpu as pltpu
from jax.experimental.pallas import tpu_sc as plsc

VOCAB = 100000
EMB = 32
HIDDEN = 512
CTX = 20
BATCH = 1024

CH = 2048
NBUF = 3
NCH = 48
TAIL = VOCAB - NCH * CH


def _sc_gather(table, idx_flat, n_rows):
    info = plsc.get_sparse_core_info()
    nw = info.num_cores * info.num_subcores
    b_per_w = n_rows // nw
    mesh = plsc.VectorSubcoreMesh(core_axis_name="c", subcore_axis_name="s")

    @functools.partial(
        pl.kernel,
        mesh=mesh,
        compiler_params=pltpu.CompilerParams(use_tc_tiling_on_sc=False),
        out_type=jax.ShapeDtypeStruct((n_rows, EMB), jnp.float32),
        scratch_types=[
            pltpu.VMEM((b_per_w,), jnp.int32),
            pltpu.VMEM((b_per_w, EMB), jnp.float32),
            pltpu.SemaphoreType.DMA,
        ],
    )
    def gather_k(idx_hbm, table_hbm, out_hbm, idx_v, rows_v, sem):
        wid = lax.axis_index("s") * info.num_cores + lax.axis_index("c")
        base = wid * b_per_w
        pltpu.sync_copy(idx_hbm.at[pl.ds(base, b_per_w)], idx_v)
        pltpu.async_copy(table_hbm.at[idx_v], rows_v, sem).wait()
        pltpu.sync_copy(rows_v, out_hbm.at[pl.ds(base, b_per_w)])

    return gather_k(idx_flat, table)


def _mlp1_body(flat_ref, w1_ref, b1_ref, h_ref):
    a = flat_ref[...].astype(jnp.bfloat16)
    w = w1_ref[...].astype(jnp.bfloat16)
    h = jnp.dot(a, w, preferred_element_type=jnp.float32)
    h_ref[...] = jnp.maximum(h + b1_ref[...], 0.0).astype(jnp.bfloat16)


def _fc2_body(h_ref, b2_ref, w2_hbm, out_hbm,
              w2_bufs, out_bufs, w2t_buf, outt_buf, in_sems, out_sems,
              t_in_sem, t_out_sem):
    def w2_copy(i, slot):
        off = pl.multiple_of(i * CH, CH)
        return pltpu.make_async_copy(
            w2_hbm.at[:, pl.ds(off, CH)], w2_bufs.at[slot], in_sems.at[slot])

    def out_copy(i, slot):
        off = pl.multiple_of(i * CH, CH)
        return pltpu.make_async_copy(
            out_bufs.at[slot], out_hbm.at[:, pl.ds(off, CH)], out_sems.at[slot])

    for s in range(NBUF):
        w2_copy(s, s).start()
    pltpu.make_async_copy(
        w2_hbm.at[:, pl.ds(NCH * CH, TAIL)], w2t_buf, t_in_sem).start()

    h = h_ref[...]

    nouter = NCH // NBUF

    def outer(i2, _):
        for b in range(NBUF):
            i = i2 * NBUF + b
            w2_copy(i, b).wait()

            @pl.when(i2 > 0)
            def _():
                out_copy(i - NBUF, b).wait()

            w2 = w2_bufs[b].astype(jnp.bfloat16)
            acc = jnp.dot(h, w2, preferred_element_type=jnp.float32)
            out_bufs[b] = acc + b2_ref[i]
            out_copy(i, b).start()

            @pl.when(i2 < nouter - 1)
            def _():
                w2_copy(i + NBUF, b).start()
        return 0

    lax.fori_loop(0, nouter, outer, 0)

    pltpu.make_async_copy(
        w2_hbm.at[:, pl.ds(NCH * CH, TAIL)], w2t_buf, t_in_sem).wait()
    acct = jnp.dot(h, w2t_buf[...].astype(jnp.bfloat16),
                   preferred_element_type=jnp.float32)
    outt_buf[...] = acct + b2_ref[NCH, :, :TAIL]
    pltpu.make_async_copy(
        outt_buf, out_hbm.at[:, pl.ds(NCH * CH, TAIL)], t_out_sem).start()

    for s in range(NBUF):
        i = NCH - NBUF + s
        out_copy(i, s).wait()
    pltpu.make_async_copy(
        outt_buf, out_hbm.at[:, pl.ds(NCH * CH, TAIL)], t_out_sem).wait()


def kernel(x, emb_table, W1, b1, W2, b2):
    idx_flat = x.reshape(-1).astype(jnp.int32)
    flat = _sc_gather(emb_table, idx_flat, BATCH * CTX)
    flat = flat.reshape(BATCH, CTX * EMB)

    h = pl.pallas_call(
        _mlp1_body,
        out_shape=jax.ShapeDtypeStruct((BATCH, HIDDEN), jnp.bfloat16),
    )(flat, W1, b1.reshape(1, HIDDEN))

    b2p = jnp.pad(b2, (0, (NCH + 1) * CH - VOCAB)).reshape(NCH + 1, 1, CH)

    logits = pl.pallas_call(
        _fc2_body,
        in_specs=[
            pl.BlockSpec(memory_space=pltpu.MemorySpace.VMEM),
            pl.BlockSpec(memory_space=pltpu.MemorySpace.VMEM),
            pl.BlockSpec(memory_space=pltpu.MemorySpace.HBM),
        ],
        out_specs=pl.BlockSpec(memory_space=pltpu.MemorySpace.HBM),
        out_shape=jax.ShapeDtypeStruct((BATCH, VOCAB), jnp.float32),
        scratch_shapes=[
            pltpu.VMEM((NBUF, HIDDEN, CH), jnp.float32),
            pltpu.VMEM((NBUF, BATCH, CH), jnp.float32),
            pltpu.VMEM((HIDDEN, TAIL), jnp.float32),
            pltpu.VMEM((BATCH, TAIL), jnp.float32),
            pltpu.SemaphoreType.DMA((NBUF,)),
            pltpu.SemaphoreType.DMA((NBUF,)),
            pltpu.SemaphoreType.DMA,
            pltpu.SemaphoreType.DMA,
        ],
    )(h, b2p, W2)

    return logits
